# Initial kernel scaffold; baseline (speedup 1.0000x reference)
import jax
import jax.numpy as jnp
from jax.experimental import pallas as pl


def kernel(x, weight, bias):
    raise NotImplementedError("write your pallas kernel here")


# speedup vs baseline: 3.0271x; 3.0271x over previous
import functools

import jax
import jax.numpy as jnp
from jax.experimental import pallas as pl
from jax.experimental.pallas import tpu as pltpu


def _linear_kernel(x_ref, w_ref, b_ref, o_ref):
    k = pl.program_id(2)
    xb = x_ref[...].astype(jnp.bfloat16)
    wb = w_ref[...].astype(jnp.bfloat16)
    part = jax.lax.dot_general(
        xb, wb, (((1,), (1,)), ((), ())),
        preferred_element_type=jnp.float32)

    @pl.when(k == 0)
    def _():
        o_ref[...] = part + b_ref[...]

    @pl.when(k != 0)
    def _():
        o_ref[...] += part


@functools.partial(jax.jit, static_argnames=())
def _linear(x2d, weight, b2):
    M, K = x2d.shape
    N = weight.shape[0]
    TM, TN, TK = 2048, 2048, 512
    grid = (M // TM, N // TN, K // TK)
    return pl.pallas_call(
        _linear_kernel,
        out_shape=jax.ShapeDtypeStruct((M, N), jnp.float32),
        grid=grid,
        in_specs=[
            pl.BlockSpec((TM, TK), lambda i, j, k: (i, k)),
            pl.BlockSpec((TN, TK), lambda i, j, k: (j, k)),
            pl.BlockSpec((1, TN), lambda i, j, k: (0, j)),
        ],
        out_specs=pl.BlockSpec((TM, TN), lambda i, j, k: (i, j)),
        compiler_params=pltpu.CompilerParams(
            dimension_semantics=("parallel", "parallel", "arbitrary"),
            vmem_limit_bytes=100 * 1024 * 1024,
        ),
        cost_estimate=pl.CostEstimate(
            flops=2 * M * N * K,
            transcendentals=0,
            bytes_accessed=(M * K + N * K + M * N) * 4,
        ),
    )(x2d, weight, b2)


def kernel(x, weight, bias):
    orig_shape = x.shape
    K = orig_shape[-1]
    N = weight.shape[0]
    x2d = x.reshape(-1, K)
    out = _linear(x2d, weight, bias.reshape(1, N))
    return out.reshape(*orig_shape[:-1], N)


# speedup vs baseline: 3.3219x; 1.0974x over previous
import functools

import jax
import jax.numpy as jnp
from jax.experimental import pallas as pl
from jax.experimental.pallas import tpu as pltpu

_TM = 256
_TN = 2048
_WCHUNK = 256


def _linear_kernel(w_hbm, x_ref, b_ref, o_ref, wb_ref, stage_ref, sem):
    j = pl.program_id(0)
    t = pl.program_id(1)

    @pl.when(t == 0)
    def _():
        base = j * _TN
        for c in range(_TN // _WCHUNK):
            cp = pltpu.make_async_copy(
                w_hbm.at[pl.ds(base + c * _WCHUNK, _WCHUNK), :],
                stage_ref,
                sem,
            )
            cp.start()
            cp.wait()
            wb_ref[pl.ds(c * _WCHUNK, _WCHUNK), :] = (
                stage_ref[...].astype(jnp.bfloat16))

    xb = x_ref[...].astype(jnp.bfloat16)
    o_ref[...] = jax.lax.dot_general(
        xb, wb_ref[...], (((1,), (1,)), ((), ())),
        preferred_element_type=jnp.float32) + b_ref[...]


@functools.partial(jax.jit, static_argnames=())
def _linear(x2d, weight, b2):
    M, K = x2d.shape
    N = weight.shape[0]
    grid = (N // _TN, M // _TM)
    return pl.pallas_call(
        _linear_kernel,
        out_shape=jax.ShapeDtypeStruct((M, N), jnp.float32),
        grid=grid,
        in_specs=[
            pl.BlockSpec(memory_space=pl.ANY),
            pl.BlockSpec((_TM, K), lambda j, t: (t, 0)),
            pl.BlockSpec((1, _TN), lambda j, t: (0, j)),
        ],
        out_specs=pl.BlockSpec((_TM, _TN), lambda j, t: (t, j)),
        scratch_shapes=[
            pltpu.VMEM((_TN, K), jnp.bfloat16),
            pltpu.VMEM((_WCHUNK, K), jnp.float32),
            pltpu.SemaphoreType.DMA,
        ],
        compiler_params=pltpu.CompilerParams(
            dimension_semantics=("parallel", "arbitrary"),
            vmem_limit_bytes=100 * 1024 * 1024,
        ),
        cost_estimate=pl.CostEstimate(
            flops=2 * M * N * K,
            transcendentals=0,
            bytes_accessed=(M * K + N * K + M * N) * 4,
        ),
    )(weight, x2d, b2)


def kernel(x, weight, bias):
    orig_shape = x.shape
    K = orig_shape[-1]
    N = weight.shape[0]
    x2d = x.reshape(-1, K)
    out = _linear(x2d, weight, bias.reshape(1, N))
    return out.reshape(*orig_shape[:-1], N)


# speedup vs baseline: 3.6246x; 1.0911x over previous
import functools

import jax
import jax.numpy as jnp
from jax.experimental import pallas as pl
from jax.experimental.pallas import tpu as pltpu

_TM = 256
_TN = 2048
_WCHUNK = 256


def _linear_kernel(w_hbm, x_ref, b_ref, o_ref, wb_ref, stage_ref, sem):
    j = pl.program_id(0)
    t = pl.program_id(1)

    @pl.when(t == 0)
    def _():
        base = j * _TN
        nchunks = _TN // _WCHUNK

        def copy(c, buf):
            return pltpu.make_async_copy(
                w_hbm.at[pl.ds(base + c * _WCHUNK, _WCHUNK), :],
                stage_ref.at[buf],
                sem.at[buf],
            )

        copy(0, 0).start()
        for c in range(nchunks):
            if c + 1 < nchunks:
                copy(c + 1, (c + 1) % 2).start()
            copy(c, c % 2).wait()
            wb_ref[pl.ds(c * _WCHUNK, _WCHUNK), :] = (
                stage_ref[c % 2].astype(jnp.bfloat16))

    xb = x_ref[...].astype(jnp.bfloat16)
    o_ref[...] = jax.lax.dot_general(
        xb, wb_ref[...], (((1,), (1,)), ((), ())),
        preferred_element_type=jnp.float32) + b_ref[...]


@functools.partial(jax.jit, static_argnames=())
def _linear(x2d, weight, b2):
    M, K = x2d.shape
    N = weight.shape[0]
    grid = (N // _TN, M // _TM)
    return pl.pallas_call(
        _linear_kernel,
        out_shape=jax.ShapeDtypeStruct((M, N), jnp.float32),
        grid=grid,
        in_specs=[
            pl.BlockSpec(memory_space=pl.ANY),
            pl.BlockSpec((_TM, K), lambda j, t: (t, 0)),
            pl.BlockSpec((1, _TN), lambda j, t: (0, j)),
        ],
        out_specs=pl.BlockSpec((_TM, _TN), lambda j, t: (t, j)),
        scratch_shapes=[
            pltpu.VMEM((_TN, K), jnp.bfloat16),
            pltpu.VMEM((2, _WCHUNK, K), jnp.float32),
            pltpu.SemaphoreType.DMA((2,)),
        ],
        compiler_params=pltpu.CompilerParams(
            dimension_semantics=("parallel", "arbitrary"),
            vmem_limit_bytes=100 * 1024 * 1024,
        ),
        cost_estimate=pl.CostEstimate(
            flops=2 * M * N * K,
            transcendentals=0,
            bytes_accessed=(M * K + N * K + M * N) * 4,
        ),
    )(weight, x2d, b2)


def kernel(x, weight, bias):
    orig_shape = x.shape
    K = orig_shape[-1]
    N = weight.shape[0]
    x2d = x.reshape(-1, K)
    out = _linear(x2d, weight, bias.reshape(1, N))
    return out.reshape(*orig_shape[:-1], N)


# speedup vs baseline: 3.6401x; 1.0043x over previous
import functools

import jax
import jax.numpy as jnp
from jax.experimental import pallas as pl
from jax.experimental.pallas import tpu as pltpu

_TM = 256
_TN = 2048
_WCHUNK = 256
_NSTAGE = 4


def _linear_kernel(w_hbm, x_ref, b_ref, o_ref, wb_ref, stage_ref, sem):
    j = pl.program_id(0)
    t = pl.program_id(1)

    @pl.when(t == 0)
    def _():
        base = j * _TN
        nchunks = _TN // _WCHUNK

        def copy(c):
            return pltpu.make_async_copy(
                w_hbm.at[pl.ds(base + c * _WCHUNK, _WCHUNK), :],
                stage_ref.at[c % _NSTAGE],
                sem.at[c % _NSTAGE],
            )

        for c in range(_NSTAGE):
            copy(c).start()
        for c in range(nchunks):
            copy(c).wait()
            wb_ref[pl.ds(c * _WCHUNK, _WCHUNK), :] = (
                stage_ref[c % _NSTAGE].astype(jnp.bfloat16))
            if c + _NSTAGE < nchunks:
                copy(c + _NSTAGE).start()

    xb = x_ref[...].astype(jnp.bfloat16)
    o_ref[...] = jax.lax.dot_general(
        xb, wb_ref[...], (((1,), (1,)), ((), ())),
        preferred_element_type=jnp.float32) + b_ref[...]


@functools.partial(jax.jit, static_argnames=())
def _linear(x2d, weight, b2):
    M, K = x2d.shape
    N = weight.shape[0]
    grid = (N // _TN, M // _TM)
    return pl.pallas_call(
        _linear_kernel,
        out_shape=jax.ShapeDtypeStruct((M, N), jnp.float32),
        grid=grid,
        in_specs=[
            pl.BlockSpec(memory_space=pl.ANY),
            pl.BlockSpec((_TM, K), lambda j, t: (t, 0)),
            pl.BlockSpec((1, _TN), lambda j, t: (0, j)),
        ],
        out_specs=pl.BlockSpec((_TM, _TN), lambda j, t: (t, j)),
        scratch_shapes=[
            pltpu.VMEM((_TN, K), jnp.bfloat16),
            pltpu.VMEM((_NSTAGE, _WCHUNK, K), jnp.float32),
            pltpu.SemaphoreType.DMA((_NSTAGE,)),
        ],
        compiler_params=pltpu.CompilerParams(
            dimension_semantics=("parallel", "arbitrary"),
            vmem_limit_bytes=100 * 1024 * 1024,
        ),
        cost_estimate=pl.CostEstimate(
            flops=2 * M * N * K,
            transcendentals=0,
            bytes_accessed=(M * K + N * K + M * N) * 4,
        ),
    )(weight, x2d, b2)


def kernel(x, weight, bias):
    orig_shape = x.shape
    K = orig_shape[-1]
    N = weight.shape[0]
    x2d = x.reshape(-1, K)
    out = _linear(x2d, weight, bias.reshape(1, N))
    return out.reshape(*orig_shape[:-1], N)


# speedup vs baseline: 3.8640x; 1.0615x over previous
import functools

import jax
import jax.numpy as jnp
from jax.experimental import pallas as pl
from jax.experimental.pallas import tpu as pltpu

_TM = 256
_TN = 2048
_WCHUNK = 256
_NSTAGE = 2


def _linear_kernel(w_hbm, x_ref, b_ref, o_ref, wb_ref, stage_ref, sem):
    j = pl.program_id(0)
    t = pl.program_id(1)
    nchunks = _TN // _WCHUNK

    def copy(half, c, buf):
        return pltpu.make_async_copy(
            w_hbm.at[pl.ds(half * _TN + c * _WCHUNK, _WCHUNK), :],
            stage_ref.at[buf],
            sem.at[buf],
        )

    def cast(half, c, buf):
        wb_ref[half, pl.ds(c * _WCHUNK, _WCHUNK), :] = (
            stage_ref[buf].astype(jnp.bfloat16))

    @pl.when((j == 0) & (t == 0))
    def _():
        copy(0, 0, 0).start()
        for c in range(nchunks):
            if c + 1 < nchunks:
                copy(0, c + 1, (c + 1) % 2).start()
            copy(0, c, c % 2).wait()
            cast(0, c, c % 2)

    for c in range(nchunks):
        @pl.when((j == 0) & (t == c + 1))
        def _(c=c):
            copy(1, c, c % 2).start()

        @pl.when((j == 0) & (t == c + 2))
        def _(c=c):
            copy(1, c, c % 2).wait()
            cast(1, c, c % 2)

    xb = x_ref[...].astype(jnp.bfloat16)
    o_ref[...] = jax.lax.dot_general(
        xb, wb_ref[j], (((1,), (1,)), ((), ())),
        preferred_element_type=jnp.float32) + b_ref[...]


@functools.partial(jax.jit, static_argnames=())
def _linear(x2d, weight, b2):
    M, K = x2d.shape
    N = weight.shape[0]
    grid = (N // _TN, M // _TM)
    return pl.pallas_call(
        _linear_kernel,
        out_shape=jax.ShapeDtypeStruct((M, N), jnp.float32),
        grid=grid,
        in_specs=[
            pl.BlockSpec(memory_space=pl.ANY),
            pl.BlockSpec((_TM, K), lambda j, t: (t, 0)),
            pl.BlockSpec((1, _TN), lambda j, t: (0, j)),
        ],
        out_specs=pl.BlockSpec((_TM, _TN), lambda j, t: (t, j)),
        scratch_shapes=[
            pltpu.VMEM((2, _TN, K), jnp.bfloat16),
            pltpu.VMEM((_NSTAGE, _WCHUNK, K), jnp.float32),
            pltpu.SemaphoreType.DMA((_NSTAGE,)),
        ],
        compiler_params=pltpu.CompilerParams(
            dimension_semantics=("arbitrary", "arbitrary"),
            vmem_limit_bytes=100 * 1024 * 1024,
        ),
        cost_estimate=pl.CostEstimate(
            flops=2 * M * N * K,
            transcendentals=0,
            bytes_accessed=(M * K + N * K + M * N) * 4,
        ),
    )(weight, x2d, b2)


def kernel(x, weight, bias):
    orig_shape = x.shape
    K = orig_shape[-1]
    N = weight.shape[0]
    x2d = x.reshape(-1, K)
    out = _linear(x2d, weight, bias.reshape(1, N))
    return out.reshape(*orig_shape[:-1], N)
